# baseline (device time: 52171 ns/iter reference)
import jax
import jax.numpy as jnp
from jax import lax
from jax.experimental import pallas as pl
from jax.experimental.pallas import tpu as pltpu

N_DEV = 4
N_LAYERS = 3

_AG_P1, _AG_P2, _AG_DIAG = 0, 1, 2
def _ar(l, s, q):
    return 3 + 6 * l + 2 * s + q


def kernel(x, Win0, Wout0, Win1, Wout1, Win2, Wout2):
    m_per, d = x.shape
    M = N_DEV * m_per
    half = M // 2
    quart = half // 2
    n_ex = 3 + 6 * N_LAYERS

    def body(x_ref, win0_ref, wout0_ref, win1_ref, wout1_ref, win2_ref,
             wout2_ref, out_ref, X_ref, acc_ref, s1_ref, r1_ref, r2_ref,
             wi32_ref, wo32_ref, wi_ref, wo_ref, wdma_sems,
             send_sems, recv_sems):
        me = lax.axis_index("i")
        p1 = me ^ 1
        p2 = 3 - me
        diag = me ^ 2

        kh0 = jnp.where((me == 0) | (me == 3), 1, 0)

        wins = (win0_ref, win1_ref, win2_ref)
        wouts = (wout0_ref, wout1_ref, wout2_ref)

        def exchange(idx, src, dst, partner):
            rdma = pltpu.make_async_remote_copy(
                src_ref=src, dst_ref=dst,
                send_sem=send_sems.at[idx], recv_sem=recv_sems.at[idx],
                device_id=(partner,), device_id_type=pl.DeviceIdType.MESH,
            )
            rdma.start()
            return rdma

        def fetch_weights(l):
            wi = pltpu.make_async_copy(wins[l], wi32_ref, wdma_sems.at[0])
            wo = pltpu.make_async_copy(wouts[l], wo32_ref, wdma_sems.at[1])
            wi.start()
            wo.start()
            return wi, wo

        def land_weights(dmas):
            for dma in dmas:
                dma.wait()
            wi_ref[...] = wi32_ref[...].astype(jnp.bfloat16)
            wo_ref[...] = wo32_ref[...].astype(jnp.bfloat16)

        def qpart(off):
            xq = X_ref[pl.ds(off, quart), :]
            h = jnp.maximum(
                jnp.dot(xq, wi_ref[...], preferred_element_type=jnp.float32),
                0.0).astype(jnp.bfloat16)
            return jnp.dot(h, wo_ref[...], preferred_element_type=jnp.float32)

        barrier = pltpu.get_barrier_semaphore()
        for nbr in (p1, p2, diag):
            pl.semaphore_signal(
                barrier, inc=1,
                device_id=(nbr,), device_id_type=pl.DeviceIdType.MESH,
            )
        pl.semaphore_wait(barrier, 3)

        X_ref[pl.ds(me * m_per, m_per), :] = x_ref[...].astype(jnp.bfloat16)
        my_sl = pl.ds(me * m_per, m_per)
        ag = [exchange(idx, X_ref.at[my_sl, :], X_ref.at[my_sl, :], tgt)
              for idx, tgt in ((_AG_P1, p1), (_AG_P2, p2), (_AG_DIAG, diag))]
        dmas = fetch_weights(0)
        land_weights(dmas)
        for rdma in ag:
            rdma.wait()

        soff0 = (1 - kh0) * half
        s1_ref[0, 0] = qpart(soff0).astype(jnp.bfloat16)
        a1 = [exchange(_ar(0, 0, 0), s1_ref.at[0, 0], r1_ref.at[0, 0], p1)]
        s1_ref[0, 1] = qpart(soff0 + quart).astype(jnp.bfloat16)
        a1.append(exchange(_ar(0, 0, 1), s1_ref.at[0, 1], r1_ref.at[0, 1], p1))

        prev_ar3 = None
        for l in range(N_LAYERS):
            kh = kh0 ^ (l & 1)
            koff = kh * half
            q_sl = (pl.ds(koff, quart), pl.ds(koff + quart, quart))

            if prev_ar3 is not None:
                for rdma in prev_ar3:
                    rdma.wait()
            acc_ref[q_sl[0], :] = qpart(koff)
            acc_ref[q_sl[1], :] = qpart(koff + quart)
            if l < N_LAYERS - 1:
                dmas = fetch_weights(l + 1)

            a2 = []
            for q in range(2):
                a1[q].wait()
                X_ref[q_sl[q], :] = (
                    acc_ref[q_sl[q], :] + r1_ref[l, q].astype(jnp.float32)
                ).astype(jnp.bfloat16)
                a2.append(exchange(
                    _ar(l, 1, q), X_ref.at[q_sl[q], :], r2_ref.at[l, q], p2))
            if l < N_LAYERS - 1:
                land_weights(dmas)
            a3 = []
            for q in range(2):
                a2[q].wait()
                if l < N_LAYERS - 1:
                    X_ref[q_sl[q], :] = (
                        acc_ref[q_sl[q], :]
                        + r1_ref[l, q].astype(jnp.float32)
                        + r2_ref[l, q].astype(jnp.float32)
                    ).astype(jnp.bfloat16)
                else:
                    acc_ref[q_sl[q], :] += (
                        r1_ref[l, q].astype(jnp.float32)
                        + r2_ref[l, q].astype(jnp.float32))
                    X_ref[q_sl[q], :] = acc_ref[q_sl[q], :].astype(jnp.bfloat16)
                a3.append(exchange(
                    _ar(l, 2, q), X_ref.at[q_sl[q], :], X_ref.at[q_sl[q], :],
                    p1))
            if l < N_LAYERS - 1:
                s1_ref[l + 1, 0] = qpart(koff).astype(jnp.bfloat16)
                a1 = [exchange(_ar(l + 1, 0, 0), s1_ref.at[l + 1, 0],
                               r1_ref.at[l + 1, 0], p1)]
                s1_ref[l + 1, 1] = qpart(koff + quart).astype(jnp.bfloat16)
                a1.append(exchange(_ar(l + 1, 0, 1), s1_ref.at[l + 1, 1],
                                   r1_ref.at[l + 1, 1], p1))
                prev_ar3 = a3
            else:
                out_ref[pl.ds(koff, half), :] = acc_ref[pl.ds(koff, half), :]
                for rdma in a3:
                    rdma.wait()
                osl = pl.ds((1 - kh) * half, half)
                out_ref[osl, :] = X_ref[osl, :].astype(jnp.float32)

    return pl.pallas_call(
        body,
        out_shape=jax.ShapeDtypeStruct((M, d), jnp.float32),
        in_specs=[pl.BlockSpec(memory_space=pltpu.VMEM)]
        + [pl.BlockSpec(memory_space=pl.ANY)] * 6,
        out_specs=pl.BlockSpec(memory_space=pltpu.VMEM),
        scratch_shapes=[
            pltpu.VMEM((M, d), jnp.bfloat16),
            pltpu.VMEM((M, d), jnp.float32),
            pltpu.VMEM((N_LAYERS, 2, quart, d), jnp.bfloat16),
            pltpu.VMEM((N_LAYERS, 2, quart, d), jnp.bfloat16),
            pltpu.VMEM((N_LAYERS, 2, quart, d), jnp.bfloat16),
            pltpu.VMEM(Win0.shape, jnp.float32),
            pltpu.VMEM(Wout0.shape, jnp.float32),
            pltpu.VMEM(Win0.shape, jnp.bfloat16),
            pltpu.VMEM(Wout0.shape, jnp.bfloat16),
            pltpu.SemaphoreType.DMA((2,)),
            pltpu.SemaphoreType.DMA((n_ex,)),
            pltpu.SemaphoreType.DMA((n_ex,)),
        ],
        compiler_params=pltpu.CompilerParams(
            collective_id=0, vmem_limit_bytes=100 * 1024 * 1024),
    )(x, Win0, Wout0, Win1, Wout1, Win2, Wout2)
